# Initial kernel scaffold; baseline (speedup 1.0000x reference)
import jax
import jax.numpy as jnp
from jax.experimental import pallas as pl


def kernel(state, W1, b1, W2, b2, Wfc, bfc):
    raise NotImplementedError("write your pallas kernel here")


# speedup vs baseline: 122.4668x; 122.4668x over previous
import jax
import jax.numpy as jnp
import numpy as np
from jax.experimental import pallas as pl

_C = float(0.5 / np.sqrt(2.0))


def _lrelu(x):
    return jnp.where(x > 0, x, 0.01 * x)


def _gnn_kernel(state_ref, w1p_ref, w2_ref, wfc_ref, b1_ref, b2_ref,
                bfc_ref, out_ref):
    st = state_ref[...]
    bb = st.shape[0]

    y = jnp.dot(st, w1p_ref[...], preferred_element_type=jnp.float32)
    y1, y2, y3 = y[:, 0:64], y[:, 64:128], y[:, 128:192]
    b1 = b1_ref[...]
    h = jnp.concatenate([
        _lrelu(_C * (y1 + y2 + y3) + b1),
        _lrelu(0.5 * y1 + b1),
        _lrelu(0.5 * y2 + b1),
        _lrelu(0.5 * y3 + b1),
    ], axis=0)

    z = jnp.dot(h, w2_ref[...], preferred_element_type=jnp.float32)
    z0, z1, z2, z3 = (z[j * bb:(j + 1) * bb] for j in range(4))
    b2 = b2_ref[...]
    g = jnp.concatenate([
        _lrelu(0.25 * z0 + _C * (z1 + z2 + z3) + b2),
        _lrelu(_C * z0 + 0.5 * z1 + b2),
        _lrelu(_C * z0 + 0.5 * z2 + b2),
        _lrelu(_C * z0 + 0.5 * z3 + b2),
    ], axis=0)

    u = jnp.dot(g, wfc_ref[...], preferred_element_type=jnp.float32)
    h3 = _lrelu(u + bfc_ref[...])
    out_ref[...] = 0.25 * (h3[0:bb] + h3[bb:2 * bb] +
                           h3[2 * bb:3 * bb] + h3[3 * bb:4 * bb])


def _pick_block(b):
    for bb in (1000, 512, 500, 256, 200, 128, 104, 64, 40, 32, 16, 8):
        if b % bb == 0:
            return bb
    return None


def kernel(state, W1, b1, W2, b2, Wfc, bfc):
    b = state.shape[0]
    bb = _pick_block(b)
    pad = 0
    if bb is None:
        bb = 512
        pad = (-b) % bb
        state = jnp.pad(state, ((0, pad), (0, 0)))
    bt = b + pad

    w1p = jnp.zeros((12, 192), jnp.float32)
    w1p = (w1p.at[0:4, 0:64].set(W1)
               .at[4:8, 64:128].set(W1)
               .at[8:12, 128:192].set(W1))

    out = pl.pallas_call(
        _gnn_kernel,
        grid=(bt // bb,),
        in_specs=[
            pl.BlockSpec((bb, 12), lambda i: (i, 0)),
            pl.BlockSpec((12, 192), lambda i: (0, 0)),
            pl.BlockSpec((64, 64), lambda i: (0, 0)),
            pl.BlockSpec((64, 256), lambda i: (0, 0)),
            pl.BlockSpec((1, 64), lambda i: (0, 0)),
            pl.BlockSpec((1, 64), lambda i: (0, 0)),
            pl.BlockSpec((1, 256), lambda i: (0, 0)),
        ],
        out_specs=pl.BlockSpec((bb, 256), lambda i: (i, 0)),
        out_shape=jax.ShapeDtypeStruct((bt, 256), jnp.float32),
    )(state, w1p, W2, Wfc,
      b1.reshape(1, 64), b2.reshape(1, 64), bfc.reshape(1, 256))
    if pad:
        out = out[:b]
    return out


# speedup vs baseline: 152.1198x; 1.2421x over previous
import jax
import jax.numpy as jnp
import numpy as np
from jax.experimental import pallas as pl

_C = float(0.5 / np.sqrt(2.0))
_A = np.array([
    [0.25, _C, _C, _C],
    [_C, 0.5, 0.0, 0.0],
    [_C, 0.0, 0.5, 0.0],
    [_C, 0.0, 0.0, 0.5],
], dtype=np.float32)


def _lrelu(x):
    return jnp.maximum(x, 0.01 * x)


def _gnn_kernel(state_ref, w1f_ref, w2f_ref, wfc_ref, b1t_ref, b2t_ref,
                bfc_ref, out_ref):
    st = state_ref[...]

    y = jnp.dot(st, w1f_ref[...], preferred_element_type=jnp.float32)
    h1 = _lrelu(y + b1t_ref[...])

    z = jnp.dot(h1, w2f_ref[...], preferred_element_type=jnp.float32)
    h2 = _lrelu(z + b2t_ref[...])

    wfc = wfc_ref[...]
    bfc = bfc_ref[...]
    acc = _lrelu(jnp.dot(h2[:, 0:64], wfc,
                         preferred_element_type=jnp.float32) + bfc)
    for j in range(1, 4):
        acc = acc + _lrelu(
            jnp.dot(h2[:, 64 * j:64 * j + 64], wfc,
                    preferred_element_type=jnp.float32) + bfc)
    out_ref[...] = 0.25 * acc


def _pick_block(b):
    for bb in (1000, 512, 500, 256, 200, 128, 104, 64, 40, 32, 16, 8):
        if b % bb == 0:
            return bb
    return None


def kernel(state, W1, b1, W2, b2, Wfc, bfc):
    b = state.shape[0]
    bb = _pick_block(b)
    pad = 0
    if bb is None:
        bb = 512
        pad = (-b) % bb
        state = jnp.pad(state, ((0, pad), (0, 0)))
    bt = b + pad

    a = jnp.asarray(_A)
    w1f = jnp.kron(a.T[1:4, :], W1)
    w2f = jnp.kron(a.T, W2)
    b1t = jnp.tile(b1, 4).reshape(1, 256)
    b2t = jnp.tile(b2, 4).reshape(1, 256)

    out = pl.pallas_call(
        _gnn_kernel,
        grid=(bt // bb,),
        in_specs=[
            pl.BlockSpec((bb, 12), lambda i: (i, 0)),
            pl.BlockSpec((12, 256), lambda i: (0, 0)),
            pl.BlockSpec((256, 256), lambda i: (0, 0)),
            pl.BlockSpec((64, 256), lambda i: (0, 0)),
            pl.BlockSpec((1, 256), lambda i: (0, 0)),
            pl.BlockSpec((1, 256), lambda i: (0, 0)),
            pl.BlockSpec((1, 256), lambda i: (0, 0)),
        ],
        out_specs=pl.BlockSpec((bb, 256), lambda i: (i, 0)),
        out_shape=jax.ShapeDtypeStruct((bt, 256), jnp.float32),
    )(state, w1f, w2f, Wfc, b1t, b2t, bfc.reshape(1, 256))
    if pad:
        out = out[:b]
    return out
